# baseline (speedup 1.0000x reference)
import functools

import jax
import jax.numpy as jnp
from jax.experimental import pallas as pl

_BILATERAL = True
_THETA_ALPHA = 8.0
_THETA_BETA = 0.125
_THETA_GAMMA = 3.0

_BI = 256
_NHALF_LOG2E = -0.5 * 1.4426950408889634


def _feats(img, n0, n, width):
    idx = n0 + jax.lax.broadcasted_iota(jnp.int32, (1, n), 1)
    y = idx // width
    x = idx - y * width
    if _BILATERAL:
        pos = jnp.concatenate([x.astype(jnp.float32),
                               y.astype(jnp.float32)], axis=0)
        return jnp.concatenate(
            [pos * (1.0 / _THETA_ALPHA), img * (1.0 / _THETA_BETA)], axis=0)
    return jnp.concatenate([x.astype(jnp.float32),
                            y.astype(jnp.float32)],
                           axis=0) * (1.0 / _THETA_GAMMA)


def _gauss_tile(img_blk_ref, img_all_ref, v_ref, out_ref, *, width):
    i = pl.program_id(1)
    n = img_all_ref.shape[2]
    bi = img_blk_ref.shape[2]
    fi = _feats(img_blk_ref[0], i * bi, bi, width)
    fa = _feats(img_all_ref[0], 0, n, width)
    vb = v_ref[0]
    sqi = jnp.sum(fi * fi, axis=0)
    sqa = jnp.sum(fa * fa, axis=0)
    g = jax.lax.dot_general(
        fi, fa, (((0,), (0,)), ((), ())),
        preferred_element_type=jnp.float32)
    d2 = (sqi[:, None] + sqa[None, :]) - 2.0 * g
    k_mat = jnp.exp2(_NHALF_LOG2E * jnp.maximum(d2, 0.0))
    out_ref[0] = jax.lax.dot_general(
        vb, k_mat, (((1,), (1,)), ((), ())),
        preferred_element_type=jnp.float32)


@jax.jit
def kernel(cur_state, input_image):
    B, C, H, W = cur_state.shape
    N = H * W

    img = input_image.reshape(B, 3, N)
    v = cur_state.reshape(B, C, N)

    out = pl.pallas_call(
        functools.partial(_gauss_tile, width=W),
        grid=(B, N // _BI),
        in_specs=[
            pl.BlockSpec((1, 3, _BI), lambda b, i: (b, 0, i)),
            pl.BlockSpec((1, 3, N), lambda b, i: (b, 0, 0)),
            pl.BlockSpec((1, C, N), lambda b, i: (b, 0, 0)),
        ],
        out_specs=pl.BlockSpec((1, C, _BI), lambda b, i: (b, 0, i)),
        out_shape=jax.ShapeDtypeStruct((B, C, N), jnp.float32),
    )(img, img, v)

    return out.reshape(B, C, H, W)


# speedup vs baseline: 1.0496x; 1.0496x over previous
import jax
import jax.numpy as jnp
from jax.experimental import pallas as pl

_BILATERAL = True
_THETA_ALPHA = 8.0
_THETA_BETA = 0.125
_THETA_GAMMA = 3.0

_BI = 640
_NHALF_LOG2E = -0.5 * 1.4426950408889634


def _gauss_tile(f_blk_ref, f_all_ref, v_ref, out_ref):
    fi = f_blk_ref[0]
    fa = f_all_ref[0]
    vb = v_ref[0]
    sqi = jnp.sum(fi * fi, axis=0)
    sqa = jnp.sum(fa * fa, axis=0)
    g = jax.lax.dot_general(
        fi, fa, (((0,), (0,)), ((), ())),
        preferred_element_type=jnp.float32)
    d2 = (sqi[:, None] + sqa[None, :]) - 2.0 * g
    k_mat = jnp.exp2(_NHALF_LOG2E * jnp.maximum(d2, 0.0))
    out_ref[0] = jax.lax.dot_general(
        vb, k_mat, (((1,), (1,)), ((), ())),
        preferred_element_type=jnp.float32)


@jax.jit
def kernel(cur_state, input_image):
    B, C, H, W = cur_state.shape
    N = H * W

    yy = jax.lax.broadcasted_iota(jnp.float32, (H, W), 0)
    xx = jax.lax.broadcasted_iota(jnp.float32, (H, W), 1)
    if _BILATERAL:
        pos = jnp.stack([xx, yy], axis=0).reshape(2, N) / _THETA_ALPHA
        col = input_image.reshape(B, 3, N) / _THETA_BETA
        feats = jnp.concatenate(
            [jnp.broadcast_to(pos[None], (B, 2, N)), col,
             jnp.zeros((B, 3, N), jnp.float32)], axis=1)
    else:
        pos = jnp.stack([xx, yy], axis=0).reshape(2, N) / _THETA_GAMMA
        feats = jnp.concatenate(
            [jnp.broadcast_to(pos[None], (B, 2, N)),
             jnp.zeros((B, 6, N), jnp.float32)], axis=1)

    v = cur_state.reshape(B, C, N)

    out = pl.pallas_call(
        _gauss_tile,
        grid=(B, N // _BI),
        in_specs=[
            pl.BlockSpec((1, 8, _BI), lambda b, i: (b, 0, i)),
            pl.BlockSpec((1, 8, N), lambda b, i: (b, 0, 0)),
            pl.BlockSpec((1, C, N), lambda b, i: (b, 0, 0)),
        ],
        out_specs=pl.BlockSpec((1, C, _BI), lambda b, i: (b, 0, i)),
        out_shape=jax.ShapeDtypeStruct((B, C, N), jnp.float32),
    )(feats, feats, v)

    return out.reshape(B, C, H, W)


# speedup vs baseline: 1.2758x; 1.2155x over previous
import functools

import jax
import jax.numpy as jnp
from jax.experimental import pallas as pl

_BILATERAL = True
_THETA_ALPHA = 8.0
_THETA_BETA = 0.125
_THETA_GAMMA = 3.0

_NB = 5
_INV2 = 3
_NHALF_LOG2E = -0.5 * 1.4426950408889634


def _pair(c, s):
    return (_INV2 * c - s) % _NB, (_INV2 * c + s) % _NB


def _gauss_pair(fi_ref, fj_ref, v_ref, out_ref, *, bi_sz):
    c = pl.program_id(1)
    s = pl.program_id(2)
    bi, bj = _pair(c, s)

    @pl.when(jnp.logical_and(c == 0, s == 0))
    def _init():
        out_ref[0] = jnp.zeros_like(out_ref[0])

    fi = fi_ref[0]
    fj = fj_ref[0]
    sqi = jnp.sum(fi * fi, axis=0)
    sqj = jnp.sum(fj * fj, axis=0)
    g = jax.lax.dot_general(
        fi, fj, (((0,), (0,)), ((), ())),
        preferred_element_type=jnp.float32)
    d2 = (sqi[:, None] + sqj[None, :]) - 2.0 * g
    k_mat = jnp.exp2(_NHALF_LOG2E * jnp.maximum(d2, 0.0))

    vj = v_ref[0, :, pl.ds(bj * bi_sz, bi_sz)]
    out_ref[0, :, pl.ds(bi * bi_sz, bi_sz)] += jax.lax.dot_general(
        vj, k_mat, (((1,), (1,)), ((), ())),
        preferred_element_type=jnp.float32)

    @pl.when(s != 0)
    def _upper():
        vi = v_ref[0, :, pl.ds(bi * bi_sz, bi_sz)]
        out_ref[0, :, pl.ds(bj * bi_sz, bi_sz)] += jax.lax.dot_general(
            vi, k_mat, (((1,), (0,)), ((), ())),
            preferred_element_type=jnp.float32)


@jax.jit
def kernel(cur_state, input_image):
    B, C, H, W = cur_state.shape
    N = H * W
    bi_sz = N // _NB

    yy = jax.lax.broadcasted_iota(jnp.float32, (H, W), 0)
    xx = jax.lax.broadcasted_iota(jnp.float32, (H, W), 1)
    if _BILATERAL:
        pos = jnp.stack([xx, yy], axis=0).reshape(2, N) / _THETA_ALPHA
        col = input_image.reshape(B, 3, N) / _THETA_BETA
        feats = jnp.concatenate(
            [jnp.broadcast_to(pos[None], (B, 2, N)), col,
             jnp.zeros((B, 3, N), jnp.float32)], axis=1)
    else:
        pos = jnp.stack([xx, yy], axis=0).reshape(2, N) / _THETA_GAMMA
        feats = jnp.concatenate(
            [jnp.broadcast_to(pos[None], (B, 2, N)),
             jnp.zeros((B, 6, N), jnp.float32)], axis=1)

    v = cur_state.reshape(B, C, N)

    out = pl.pallas_call(
        functools.partial(_gauss_pair, bi_sz=bi_sz),
        grid=(B, _NB, (_NB + 1) // 2),
        in_specs=[
            pl.BlockSpec((1, 8, bi_sz), lambda b, c, s: (b, 0, _pair(c, s)[0])),
            pl.BlockSpec((1, 8, bi_sz), lambda b, c, s: (b, 0, _pair(c, s)[1])),
            pl.BlockSpec((1, C, N), lambda b, c, s: (b, 0, 0)),
        ],
        out_specs=pl.BlockSpec((1, C, N), lambda b, c, s: (b, 0, 0)),
        out_shape=jax.ShapeDtypeStruct((B, C, N), jnp.float32),
    )(feats, feats, v)

    return out.reshape(B, C, H, W)


# speedup vs baseline: 1.3152x; 1.0310x over previous
import functools

import jax
import jax.numpy as jnp
from jax.experimental import pallas as pl

_BILATERAL = True
_THETA_ALPHA = 8.0
_THETA_BETA = 0.125
_THETA_GAMMA = 3.0

_NB = 5
_INV2 = 3
_LOG2E = 1.4426950408889634
_NHALF_LOG2E = -0.5 * _LOG2E


def _pair(c, s):
    return (_INV2 * c - s) % _NB, (_INV2 * c + s) % _NB


def _gauss_pair(fi_ref, fj_ref, v_ref, out_ref, *, bi_sz):
    c = pl.program_id(1)
    s = pl.program_id(2)
    bi, bj = _pair(c, s)

    @pl.when(jnp.logical_and(c == 0, s == 0))
    def _init():
        out_ref[0] = jnp.zeros_like(out_ref[0])

    fi = fi_ref[0]
    fj = fj_ref[0]
    ci = _NHALF_LOG2E * jnp.sum(fi * fi, axis=0)
    cj = _NHALF_LOG2E * jnp.sum(fj * fj, axis=0)
    g = jax.lax.dot_general(
        fi, fj, (((0,), (0,)), ((), ())),
        preferred_element_type=jnp.float32)
    t = (ci[:, None] + cj[None, :]) + _LOG2E * g
    k_mat = jnp.exp2(jnp.minimum(t, 0.0))

    vj = v_ref[0, :, pl.ds(bj * bi_sz, bi_sz)]
    out_ref[0, :, pl.ds(bi * bi_sz, bi_sz)] += jax.lax.dot_general(
        vj, k_mat, (((1,), (1,)), ((), ())),
        preferred_element_type=jnp.float32)

    @pl.when(s != 0)
    def _upper():
        vi = v_ref[0, :, pl.ds(bi * bi_sz, bi_sz)]
        out_ref[0, :, pl.ds(bj * bi_sz, bi_sz)] += jax.lax.dot_general(
            vi, k_mat, (((1,), (0,)), ((), ())),
            preferred_element_type=jnp.float32)


@jax.jit
def kernel(cur_state, input_image):
    B, C, H, W = cur_state.shape
    N = H * W
    bi_sz = N // _NB

    yy = jax.lax.broadcasted_iota(jnp.float32, (H, W), 0)
    xx = jax.lax.broadcasted_iota(jnp.float32, (H, W), 1)
    if _BILATERAL:
        pos = jnp.stack([xx, yy], axis=0).reshape(2, N) / _THETA_ALPHA
        col = input_image.reshape(B, 3, N) / _THETA_BETA
        feats = jnp.concatenate(
            [jnp.broadcast_to(pos[None], (B, 2, N)), col,
             jnp.zeros((B, 3, N), jnp.float32)], axis=1)
    else:
        pos = jnp.stack([xx, yy], axis=0).reshape(2, N) / _THETA_GAMMA
        feats = jnp.concatenate(
            [jnp.broadcast_to(pos[None], (B, 2, N)),
             jnp.zeros((B, 6, N), jnp.float32)], axis=1)

    v = cur_state.reshape(B, C, N)

    out = pl.pallas_call(
        functools.partial(_gauss_pair, bi_sz=bi_sz),
        grid=(B, _NB, (_NB + 1) // 2),
        in_specs=[
            pl.BlockSpec((1, 8, bi_sz), lambda b, c, s: (b, 0, _pair(c, s)[0])),
            pl.BlockSpec((1, 8, bi_sz), lambda b, c, s: (b, 0, _pair(c, s)[1])),
            pl.BlockSpec((1, C, N), lambda b, c, s: (b, 0, 0)),
        ],
        out_specs=pl.BlockSpec((1, C, N), lambda b, c, s: (b, 0, 0)),
        out_shape=jax.ShapeDtypeStruct((B, C, N), jnp.float32),
    )(feats, feats, v)

    return out.reshape(B, C, H, W)
